# baseline (speedup 1.0000x reference)
import jax
import jax.numpy as jnp
from jax.experimental import pallas as pl
from jax.experimental.pallas import tpu as pltpu

_BLK_ROWS = 256


def _copy_body(layer_ref, src_ref, dst_ref):
    del layer_ref
    dst_ref[...] = src_ref[...]


def kernel(memory, layer):
    _, stm, emb = memory.shape
    layer_arr = jnp.atleast_1d(jnp.asarray(layer, dtype=jnp.int32))
    grid = stm // _BLK_ROWS
    grid_spec = pltpu.PrefetchScalarGridSpec(
        num_scalar_prefetch=1,
        grid=(grid,),
        in_specs=[
            pl.BlockSpec((1, _BLK_ROWS, emb), lambda i, layer_ref: (layer_ref[0], i, 0))
        ],
        out_specs=pl.BlockSpec((1, _BLK_ROWS, emb), lambda i, layer_ref: (0, i, 0)),
    )
    return pl.pallas_call(
        _copy_body,
        grid_spec=grid_spec,
        out_shape=jax.ShapeDtypeStruct((1, stm, emb), memory.dtype),
    )(layer_arr, memory)
